# Initial kernel scaffold; baseline (speedup 1.0000x reference)
import jax
import jax.numpy as jnp
from jax.experimental import pallas as pl


def kernel(x, sos, table_0_0, table_0_1, table_0_2, table_0_3, table_1_0, table_1_1, table_1_2, table_1_3, table_2_0, table_2_1, table_2_2, table_2_3):
    raise NotImplementedError("write your pallas kernel here")


# speedup vs baseline: 2.4631x; 2.4631x over previous
import functools

import jax
import jax.numpy as jnp
from jax import lax
from jax.experimental import pallas as pl
from jax.experimental.pallas import tpu as pltpu
from jax.experimental.pallas import tpu_sc as plsc

N_WORDS = 1000
B, S = 16, 2048
GROUP_DIMS = (512, 256, 256)
N_CB = 4
OUT_D = sum(GROUP_DIMS)

NC, NS, L = 2, 16, 16
NW = NC * NS
TOK = B * S
T_PER_W = TOK // NW

CHUNK = 64
N_CHUNK = T_PER_W // CHUNK


def _accum(acc, tmp, rows, d):
  nsl = d // L

  def body(i, carry):
    for jj in range(nsl):
      sl = pl.ds(jj * L, L)
      plsc.addupdate(acc.at[i, sl], tmp[i, sl])
    return carry

  lax.fori_loop(0, rows, body, 0)


def _sc_body(xt, sos, t00, t01, t02, t03, t10, t11, t12, t13, t20, t21,
             t22, t23, out, idx_v, acc0, tmp0, acc1, tmp1, sos_v, sem):
  tables = ((t00, t01, t02, t03), (t10, t11, t12, t13), (t20, t21, t22, t23))
  wid = lax.axis_index("s") * NC + lax.axis_index("c")
  tok0 = wid * T_PER_W
  b = wid // 2
  s0 = (wid % 2) * T_PER_W

  pltpu.sync_copy(xt.at[:, pl.ds(tok0, T_PER_W)], idx_v)

  pltpu.sync_copy(sos, sos_v)

  @pl.when(wid % 2 == 0)
  def _():
    pltpu.sync_copy(sos_v, out.at[b, 0, :])

  col = 0
  for g, d in enumerate(GROUP_DIMS):
    acc, tmp = (acc0, tmp0) if d == 512 else (acc1, tmp1)
    jbase = g * N_CB

    def chunk_body(c, carry, *, g=g, d=d, col=col, jbase=jbase, acc=acc,
                   tmp=tmp):
      off = c * CHUNK
      pltpu.async_copy(
          tables[g][0].at[idx_v.at[jbase, pl.ds(off, CHUNK)]], acc,
          sem).wait()
      for j in range(1, N_CB):
        pltpu.async_copy(
            tables[g][j].at[idx_v.at[jbase + j, pl.ds(off, CHUNK)]], tmp,
            sem).wait()
        _accum(acc, tmp, CHUNK, d)
      pltpu.sync_copy(
          acc, out.at[b, pl.ds(1 + s0 + off, CHUNK), pl.ds(col, d)])
      return carry

    lax.fori_loop(0, N_CHUNK, chunk_body, 0)
    col += d


def kernel(x, sos, table_0_0, table_0_1, table_0_2, table_0_3, table_1_0,
           table_1_1, table_1_2, table_1_3, table_2_0, table_2_1, table_2_2,
           table_2_3):
  xt = x.reshape(TOK, 12).T
  sos_flat = sos.reshape(OUT_D)

  mesh = plsc.VectorSubcoreMesh(
      core_axis_name="c", subcore_axis_name="s", num_cores=NC,
      num_subcores=NS)
  kfn = pl.kernel(
      _sc_body,
      out_type=jax.ShapeDtypeStruct((B, S + 1, OUT_D), jnp.float32),
      mesh=mesh,
      compiler_params=pltpu.CompilerParams(use_tc_tiling_on_sc=False),
      scratch_types=[
          pltpu.VMEM((12, T_PER_W), jnp.int32),
          pltpu.VMEM((CHUNK, 512), jnp.float32),
          pltpu.VMEM((CHUNK, 512), jnp.float32),
          pltpu.VMEM((CHUNK, 256), jnp.float32),
          pltpu.VMEM((CHUNK, 256), jnp.float32),
          pltpu.VMEM((OUT_D,), jnp.float32),
          pltpu.SemaphoreType.DMA,
      ],
  )
  return kfn(xt, sos_flat, table_0_0, table_0_1, table_0_2, table_0_3,
             table_1_0, table_1_1, table_1_2, table_1_3, table_2_0,
             table_2_1, table_2_2, table_2_3)


# speedup vs baseline: 3.0126x; 1.2231x over previous
import jax
import jax.numpy as jnp
from jax import lax
from jax.experimental import pallas as pl
from jax.experimental.pallas import tpu as pltpu
from jax.experimental.pallas import tpu_sc as plsc

N_WORDS = 1000
B, S = 16, 2048
GROUP_DIMS = (512, 256, 256)
N_CB = 4
OUT_D = sum(GROUP_DIMS)
N_TAB = 12

NC, NS, L = 2, 16, 16
NW = NC * NS
TOK = B * S
T_PER_W = TOK // NW

CHUNK = 32
N_CHUNK = T_PER_W // CHUNK


def _accum(acc, tmp, d):
  nsl = d // L

  def body(i, carry):
    for jj in range(nsl):
      sl = pl.ds(jj * L, L)
      plsc.addupdate(acc.at[i, sl], tmp[i, sl])
    return carry

  lax.fori_loop(0, CHUNK, body, 0)


def _sc_body(x_flat, sos, t00, t01, t02, t03, t10, t11, t12, t13, t20, t21,
             t22, t23, out, xslab, idx_v, a0a, a0b, m0a, m0b, a1a, a1b, m1a,
             m1b, sos_v, sA0a, sA0b, sT0a, sT0b, sO0a, sO0b, sA1a, sA1b,
             sT1a, sT1b, sO1a, sO1b):
  group_tabs = ((t00, t01, t02, t03), (t10, t11, t12, t13),
                (t20, t21, t22, t23))
  wid = lax.axis_index("s") * NC + lax.axis_index("c")
  tok0 = wid * T_PER_W
  b = wid // 2
  s0 = (wid % 2) * T_PER_W

  pltpu.sync_copy(x_flat.at[pl.ds(tok0 * N_TAB, T_PER_W * N_TAB)], xslab)
  iota12 = lax.iota(jnp.int32, L) * N_TAB

  def tbody(k, carry):
    rows12 = iota12 + k * (L * N_TAB)
    for j in range(N_TAB):
      idx_v[j, pl.ds(k * L, L)] = plsc.load_gather(xslab, [rows12 + j])
    return carry

  lax.fori_loop(0, T_PER_W // L, tbody, 0)

  pltpu.sync_copy(sos, sos_v)

  @pl.when(wid % 2 == 0)
  def _():
    pltpu.sync_copy(sos_v, out.at[b, 0, :])

  def run_group(tabs, d, col, jbase, accs, tmps, sA, sT, sO,
                drain_last=True):
    def gidx(c, j):
      return idx_v.at[jbase + j, pl.ds(c * CHUNK, CHUNK)]

    def gather(j, c, buf, sem):
      pltpu.async_copy(tabs[j].at[gidx(c, j)], buf, sem)

    def wait_gather(buf, sem):
      pltpu.make_async_copy(tabs[0].at[pl.ds(0, CHUNK)], buf, sem).wait()

    def out_dst(c):
      return out.at[b, pl.ds(1 + s0 + c * CHUNK, CHUNK), pl.ds(col, d)]

    def wait_out(p):
      pltpu.make_async_copy(accs[p], out_dst(0), sO[p]).wait()

    def do_chunk(c, p):
      q = 1 - p
      acc = accs[p]

      @pl.when(c >= 1)
      def _():
        wait_out(q)

      @pl.when(c < N_CHUNK - 1)
      def _():
        gather(0, c + 1, accs[q], sA[q])

      wait_gather(acc, sA[p])
      wait_gather(tmps[p], sT[p])
      gather(2, c, tmps[q], sT[q])
      _accum(acc, tmps[p], d)
      wait_gather(tmps[q], sT[q])
      gather(3, c, tmps[p], sT[p])
      _accum(acc, tmps[q], d)
      wait_gather(tmps[p], sT[p])

      @pl.when(c < N_CHUNK - 1)
      def _():
        gather(1, c + 1, tmps[q], sT[q])

      _accum(acc, tmps[p], d)
      pltpu.async_copy(acc, out_dst(c), sO[p])

    gather(0, 0, accs[0], sA[0])
    gather(1, 0, tmps[0], sT[0])

    def pair_body(c2, carry):
      do_chunk(2 * c2, 0)
      do_chunk(2 * c2 + 1, 1)
      return carry

    lax.fori_loop(0, N_CHUNK // 2, pair_body, 0)
    if drain_last:
      wait_out(1)
    return wait_out

  w0 = run_group(group_tabs[0], 512, 0, 0, (a0a, a0b), (m0a, m0b),
                 (sA0a, sA0b), (sT0a, sT0b), (sO0a, sO0b), drain_last=False)
  run_group(group_tabs[1], 256, 512, 4, (a1a, a1b), (m1a, m1b),
            (sA1a, sA1b), (sT1a, sT1b), (sO1a, sO1b))
  run_group(group_tabs[2], 256, 768, 8, (a1a, a1b), (m1a, m1b),
            (sA1a, sA1b), (sT1a, sT1b), (sO1a, sO1b))
  w0(1)


def kernel(x, sos, table_0_0, table_0_1, table_0_2, table_0_3, table_1_0,
           table_1_1, table_1_2, table_1_3, table_2_0, table_2_1, table_2_2,
           table_2_3):
  x_flat = x.reshape(TOK * N_TAB)
  sos_flat = sos.reshape(OUT_D)

  mesh = plsc.VectorSubcoreMesh(
      core_axis_name="c", subcore_axis_name="s", num_cores=NC,
      num_subcores=NS)
  kfn = pl.kernel(
      _sc_body,
      out_type=jax.ShapeDtypeStruct((B, S + 1, OUT_D), jnp.float32),
      mesh=mesh,
      compiler_params=pltpu.CompilerParams(
          use_tc_tiling_on_sc=False, needs_layout_passes=False),
      scratch_types=[
          pltpu.VMEM((T_PER_W * N_TAB,), jnp.int32),
          pltpu.VMEM((N_TAB, T_PER_W), jnp.int32),
          pltpu.VMEM((CHUNK, 512), jnp.float32),
          pltpu.VMEM((CHUNK, 512), jnp.float32),
          pltpu.VMEM((CHUNK, 512), jnp.float32),
          pltpu.VMEM((CHUNK, 512), jnp.float32),
          pltpu.VMEM((CHUNK, 256), jnp.float32),
          pltpu.VMEM((CHUNK, 256), jnp.float32),
          pltpu.VMEM((CHUNK, 256), jnp.float32),
          pltpu.VMEM((CHUNK, 256), jnp.float32),
          pltpu.VMEM((OUT_D,), jnp.float32),
      ] + [pltpu.SemaphoreType.DMA] * 12,
  )
  return kfn(x_flat, sos_flat, table_0_0, table_0_1, table_0_2, table_0_3,
             table_1_0, table_1_1, table_1_2, table_1_3, table_2_0,
             table_2_1, table_2_2, table_2_3)


# speedup vs baseline: 4.4358x; 1.4724x over previous
import jax
import jax.numpy as jnp
from jax import lax
from jax.experimental import pallas as pl
from jax.experimental.pallas import tpu as pltpu
from jax.experimental.pallas import tpu_sc as plsc

N_WORDS = 1000
B, S = 16, 2048
GROUP_DIMS = (512, 256, 256)
N_CB = 4
OUT_D = sum(GROUP_DIMS)
N_TAB = 12

NC, NS, L = 2, 16, 16
NW = NC * NS
TOK = B * S
T_PER_W = TOK // NW

CHUNK = 32
N_CHUNK = T_PER_W // CHUNK


def _accum(acc, tmp, nh):
  def body(i, carry):
    for h in range(nh):
      for jj in range(128 // L):
        sl = pl.ds(jj * L, L)
        plsc.addupdate(acc.at[i, h, sl], tmp[i, h, sl])
    return carry

  lax.fori_loop(0, CHUNK, body, 0)


def _sc_body(x_flat, sos, t00, t01, t02, t03, t10, t11, t12, t13, t20, t21,
             t22, t23, out, xslab, idx_v, a0a, a0b, m0a, m0b, a1a, a1b, m1a,
             m1b, sos_v, sA0a, sA0b, sT0a, sT0b, sO0a, sO0b, sA1a, sA1b,
             sT1a, sT1b, sO1a, sO1b):
  group_tabs = ((t00, t01, t02, t03), (t10, t11, t12, t13),
                (t20, t21, t22, t23))
  wid = lax.axis_index("s") * NC + lax.axis_index("c")
  tok0 = wid * T_PER_W
  b = wid // 2
  b_hi = b // 8
  b_lo = b % 8
  s0 = (wid % 2) * T_PER_W

  pltpu.sync_copy(x_flat.at[pl.ds(tok0 * N_TAB, T_PER_W * N_TAB)], xslab)
  iota12 = lax.iota(jnp.int32, L) * N_TAB

  def tbody(k, carry):
    rows12 = iota12 + k * (L * N_TAB)
    for j in range(N_TAB):
      idx_v[j, pl.ds(k * L, L)] = plsc.load_gather(xslab, [rows12 + j])
    return carry

  lax.fori_loop(0, T_PER_W // L, tbody, 0)

  pltpu.sync_copy(sos, sos_v)

  @pl.when(wid % 2 == 0)
  def _():
    pltpu.sync_copy(sos_v, out.at[0, b_hi, :, b_lo, :])

  def run_group(tabs, nh, h0, jbase, accs, tmps, sA, sT, sO):
    def gidx(c, j):
      return idx_v.at[jbase + j, pl.ds(c * CHUNK, CHUNK)]

    def gather(j, c, buf, sem):
      pltpu.async_copy(tabs[j].at[gidx(c, j)], buf, sem)

    def wait_gather(buf, sem):
      pltpu.make_async_copy(tabs[0].at[pl.ds(0, CHUNK)], buf, sem).wait()

    def out_dst(c):
      return out.at[pl.ds(1 + s0 + c * CHUNK, CHUNK), b_hi,
                    pl.ds(h0, nh), b_lo, :]

    def wait_out(p):
      pltpu.make_async_copy(accs[p], out_dst(0), sO[p]).wait()

    def do_chunk(c, p):
      q = 1 - p
      acc = accs[p]

      @pl.when(c >= 1)
      def _():
        wait_out(q)

      @pl.when(c < N_CHUNK - 1)
      def _():
        gather(0, c + 1, accs[q], sA[q])

      wait_gather(acc, sA[p])
      wait_gather(tmps[p], sT[p])
      gather(2, c, tmps[q], sT[q])
      _accum(acc, tmps[p], nh)
      wait_gather(tmps[q], sT[q])
      gather(3, c, tmps[p], sT[p])
      _accum(acc, tmps[q], nh)
      wait_gather(tmps[p], sT[p])

      @pl.when(c < N_CHUNK - 1)
      def _():
        gather(1, c + 1, tmps[q], sT[q])

      _accum(acc, tmps[p], nh)
      pltpu.async_copy(acc, out_dst(c), sO[p])

    gather(0, 0, accs[0], sA[0])
    gather(1, 0, tmps[0], sT[0])

    def pair_body(c2, carry):
      do_chunk(2 * c2, 0)
      do_chunk(2 * c2 + 1, 1)
      return carry

    lax.fori_loop(0, N_CHUNK // 2, pair_body, 0)
    return wait_out

  w0 = run_group(group_tabs[0], 4, 0, 0, (a0a, a0b), (m0a, m0b),
                 (sA0a, sA0b), (sT0a, sT0b), (sO0a, sO0b))
  w1 = run_group(group_tabs[1], 2, 4, 4, (a1a, a1b), (m1a, m1b),
                 (sA1a, sA1b), (sT1a, sT1b), (sO1a, sO1b))
  w1(1)
  w2 = run_group(group_tabs[2], 2, 6, 8, (a1a, a1b), (m1a, m1b),
                 (sA1a, sA1b), (sT1a, sT1b), (sO1a, sO1b))
  w2(1)
  w0(1)


def kernel(x, sos, table_0_0, table_0_1, table_0_2, table_0_3, table_1_0,
           table_1_1, table_1_2, table_1_3, table_2_0, table_2_1, table_2_2,
           table_2_3):
  x_flat = x.reshape(TOK * N_TAB)
  sos_2d = sos.reshape(8, 128)
  tabs = [
      t.reshape(N_WORDS + 1, d // 128, 128)
      for t, d in zip(
          (table_0_0, table_0_1, table_0_2, table_0_3, table_1_0, table_1_1,
           table_1_2, table_1_3, table_2_0, table_2_1, table_2_2, table_2_3),
          (512,) * 4 + (256,) * 8)
  ]

  mesh = plsc.VectorSubcoreMesh(
      core_axis_name="c", subcore_axis_name="s", num_cores=NC,
      num_subcores=NS)
  kfn = pl.kernel(
      _sc_body,
      out_type=jax.ShapeDtypeStruct((S + 1, 2, 8, 8, 128), jnp.float32),
      mesh=mesh,
      compiler_params=pltpu.CompilerParams(
          use_tc_tiling_on_sc=False, needs_layout_passes=False),
      scratch_types=[
          pltpu.VMEM((T_PER_W * N_TAB,), jnp.int32),
          pltpu.VMEM((N_TAB, T_PER_W), jnp.int32),
          pltpu.VMEM((CHUNK, 4, 128), jnp.float32),
          pltpu.VMEM((CHUNK, 4, 128), jnp.float32),
          pltpu.VMEM((CHUNK, 4, 128), jnp.float32),
          pltpu.VMEM((CHUNK, 4, 128), jnp.float32),
          pltpu.VMEM((CHUNK, 2, 128), jnp.float32),
          pltpu.VMEM((CHUNK, 2, 128), jnp.float32),
          pltpu.VMEM((CHUNK, 2, 128), jnp.float32),
          pltpu.VMEM((CHUNK, 2, 128), jnp.float32),
          pltpu.VMEM((8, 128), jnp.float32),
      ] + [pltpu.SemaphoreType.DMA] * 12,
  )
  out5 = kfn(x_flat, sos_2d, *tabs)
  return out5.transpose((1, 3, 0, 2, 4)).reshape(B, S + 1, OUT_D)
